# baseline (speedup 1.0000x reference)
import functools

import jax
from jax.experimental import pallas as pl
from jax.experimental.pallas import tpu as pltpu
from jax.experimental.pallas import tpu_sc as plsc


@functools.lru_cache(maxsize=None)
def _make_sc_lookup(B, D, dtype):
    mesh = plsc.ScalarSubcoreMesh(axis_name="c", num_cores=1)

    @functools.partial(
        pl.kernel,
        mesh=mesh,
        out_type=jax.ShapeDtypeStruct((B, D), dtype),
    )
    def k(w_hbm, out_hbm):
        pltpu.sync_copy(w_hbm.at[pl.ds(0, B)], out_hbm)

    return k


def kernel(tokens, W_pos):
    B = tokens.shape[0]
    D = W_pos.shape[1]
    return _make_sc_lookup(B, D, W_pos.dtype)(W_pos)
